# Initial kernel scaffold; baseline (speedup 1.0000x reference)
import jax
import jax.numpy as jnp
from jax.experimental import pallas as pl


def kernel(key_cache, value_cache, attn, comp_key_emb, comp_value_emb):
    raise NotImplementedError("write your pallas kernel here")


# speedup vs baseline: 1.4659x; 1.4659x over previous
import functools

import jax
import jax.numpy as jnp
from jax import lax
from jax.experimental import pallas as pl
from jax.experimental.pallas import tpu as pltpu
from jax.experimental.pallas import tpu_sc as plsc

H = 12
D = 64
S = 2048
COMP = 8
NUM_RETR = 2
H_NR = H - NUM_RETR
K_KEEP = int(S * 0.3)
N_DROP = S - K_KEEP
RECENT = 128
BR = 128
NBR = S // BR
TB = 256
NTB = S // TB
N_KEPT_ROWS = K_KEEP * H_NR
N_RETR_ROWS = S * NUM_RETR
N_ROWS = 10240
ROWS_PER_TILE = N_ROWS // 32
IDX_MINOR = 80


def _attn_stats_body(a_ref, stats_ref, colsum_ref):
    r = pl.program_id(1)
    a = a_ref[0]
    rows = lax.broadcasted_iota(jnp.int32, (BR, S), 0) + r * BR
    cols = lax.broadcasted_iota(jnp.int32, (BR, S), 1)
    dist = jnp.abs(rows - cols).astype(jnp.float32)
    w = jnp.sum(a * dist)
    s = jnp.sum(a)
    lane = lax.broadcasted_iota(jnp.int32, (1, 1, 128), 2)
    contrib = jnp.where(lane == 0, w, 0.0) + jnp.where(lane == 1, s, 0.0)

    @pl.when(r == 0)
    def _():
        stats_ref[...] = contrib

    @pl.when(r > 0)
    def _():
        stats_ref[...] = stats_ref[...] + contrib

    @pl.when(r == NBR - 1)
    def _():
        acc8 = a[0:8]
        for c in range(1, BR // 8):
            acc8 = acc8 + a[c * 8:(c + 1) * 8]
        b4 = acc8[0:4] + acc8[4:8]
        b2 = b4[0:2] + b4[2:4]
        colsum_ref[...] = jnp.reshape(b2[0:1] + b2[1:2], (1, 1, S))


def _attn_stats(a3):
    return pl.pallas_call(
        _attn_stats_body,
        grid=(H, NBR),
        in_specs=[pl.BlockSpec((1, BR, S), lambda h, r: (h, r, 0))],
        out_specs=[
            pl.BlockSpec((1, 1, 128), lambda h, r: (h, 0, 0)),
            pl.BlockSpec((1, 1, S), lambda h, r: (h, 0, 0)),
        ],
        out_shape=[
            jax.ShapeDtypeStruct((H, 1, 128), jnp.float32),
            jax.ShapeDtypeStruct((H, 1, S), jnp.float32),
        ],
    )(a3)


_EXACT = lax.Precision.HIGHEST


def _headscore_body(stats_ref, cs_ref, inv_ref, gh_ref, sn_ref):
    stats = stats_ref[:, 0, :]
    wsum = stats[:, 0:1]
    asum = stats[:, 1:2]
    avg = wsum / (asum + 1e-10)
    hi = lax.broadcasted_iota(jnp.int32, (H, H), 0)
    gi = lax.broadcasted_iota(jnp.int32, (H, H), 1)
    eye = (hi == gi).astype(jnp.float32)
    avg_row = lax.dot_general(avg, eye, (((0,), (0,)), ((), ())),
                              precision=_EXACT)
    beats = (avg_row > avg) | ((avg_row == avg) & (gi < hi))
    hr = jnp.sum(beats.astype(jnp.float32), axis=1, keepdims=True)
    hr_row = lax.dot_general(hr, eye, (((0,), (0,)), ((), ())),
                             precision=_EXACT)
    inv_oh = (jnp.broadcast_to(hr_row, (H, H))
              == lax.broadcasted_iota(jnp.int32, (H, H), 0).astype(jnp.float32)
              ).astype(jnp.float32)
    hcol = lax.broadcasted_iota(jnp.int32, (H, 1), 0).astype(jnp.float32)
    inv = lax.dot_general(inv_oh, hcol, (((1,), (0,)), ((), ())),
                          precision=_EXACT)
    inv_ref[...] = jnp.broadcast_to(inv, (H, 128))
    ji = lax.broadcasted_iota(jnp.int32, (H, 128), 1)
    sigma = jnp.where(ji < H_NR, ji + NUM_RETR,
                      jnp.where(ji < H, ji - H_NR, 99)).astype(jnp.float32)
    sel = (jnp.broadcast_to(hr, (H, 128)) == sigma).astype(jnp.float32)
    gh_ref[...] = lax.dot_general(hcol, sel, (((0,), (0,)), ((), ())),
                                  precision=_EXACT)
    mi = lax.broadcasted_iota(jnp.int32, (H_NR, H), 0)
    perm = (jnp.broadcast_to(hr_row, (H_NR, H))
            == (mi + NUM_RETR).astype(jnp.float32)).astype(jnp.float32)
    t = lax.dot_general(perm, cs_ref[:, 0, :], (((1,), (0,)), ((), ())),
                        precision=_EXACT)
    pad = jnp.concatenate([t[8:10], jnp.zeros((8 - (H_NR - 8), S),
                                              jnp.float32)], axis=0)
    acc8 = t[0:8] + pad
    b4 = acc8[0:4] + acc8[4:8]
    b2 = b4[0:2] + b4[2:4]
    s1 = b2[0:1] + b2[1:2]
    smax = jnp.max(s1)
    sn_ref[...] = jnp.where(smax > 0.0, s1 / smax, s1)


def _headscore(stats, colsum):
    return pl.pallas_call(
        _headscore_body,
        grid=(1,),
        in_specs=[
            pl.BlockSpec((H, 1, 128), lambda i: (0, 0, 0)),
            pl.BlockSpec((H, 1, S), lambda i: (0, 0, 0)),
        ],
        out_specs=[
            pl.BlockSpec((H, 128), lambda i: (0, 0)),
            pl.BlockSpec((1, 128), lambda i: (0, 0)),
            pl.BlockSpec((1, S), lambda i: (0, 0)),
        ],
        out_shape=[
            jax.ShapeDtypeStruct((H, 128), jnp.float32),
            jax.ShapeDtypeStruct((1, 128), jnp.float32),
            jax.ShapeDtypeStruct((1, S), jnp.float32),
        ],
    )(stats, colsum)


def _rank_body(scol_ref, srow_ref, kc_ref, vc_ref,
               ranks_ref, sumk_ref, sumv_ref):
    r = pl.program_id(0)
    s_col = scol_ref[...]
    s_jb = srow_ref[...]
    ii = lax.broadcasted_iota(jnp.int32, (S, TB), 0)
    jj = lax.broadcasted_iota(jnp.int32, (S, TB), 1) + r * TB
    cmpc = (s_col > s_jb) | ((s_col == s_jb) & (ii < jj))
    rankrow = jnp.sum(cmpc.astype(jnp.float32), axis=0, keepdims=True)
    ranks_ref[...] = jnp.reshape(rankrow, (1, 1, TB))
    dropm = (rankrow >= float(K_KEEP)).astype(jnp.float32)
    kc = kc_ref[...]
    vc = vc_ref[...]
    sumk = lax.dot_general(dropm, kc, (((1,), (0,)), ((), ())),
                           precision=_EXACT)
    sumv = lax.dot_general(dropm, vc, (((1,), (0,)), ((), ())),
                           precision=_EXACT)
    sumk_ref[...] = jnp.reshape(sumk, (1, 1, H * D))
    sumv_ref[...] = jnp.reshape(sumv, (1, 1, H * D))


def _rank(scol, snrow, kcf, vcf):
    return pl.pallas_call(
        _rank_body,
        grid=(NTB,),
        in_specs=[
            pl.BlockSpec((S, 1), lambda r: (0, 0)),
            pl.BlockSpec((1, TB), lambda r: (0, r)),
            pl.BlockSpec((TB, H * D), lambda r: (r, 0)),
            pl.BlockSpec((TB, H * D), lambda r: (r, 0)),
        ],
        out_specs=[
            pl.BlockSpec((1, 1, TB), lambda r: (r, 0, 0)),
            pl.BlockSpec((1, 1, H * D), lambda r: (r, 0, 0)),
            pl.BlockSpec((1, 1, H * D), lambda r: (r, 0, 0)),
        ],
        out_shape=[
            jax.ShapeDtypeStruct((NTB, 1, TB), jnp.float32),
            jax.ShapeDtypeStruct((NTB, 1, H * D), jnp.float32),
            jax.ShapeDtypeStruct((NTB, 1, H * D), jnp.float32),
        ],
    )(scol, snrow, kcf, vcf)


def _glist_body(rr_ref, gh_ref, g_ref):
    b = pl.program_id(0)
    rrow = rr_ref[...]
    ri = (lax.broadcasted_iota(jnp.int32, (BR, S), 0)
          + b * BR).astype(jnp.float32)
    oh = (jnp.broadcast_to(rrow, (BR, S)) == ri).astype(jnp.float32)
    tid = lax.broadcasted_iota(jnp.int32, (BR, S), 1).astype(jnp.float32)
    kt = jnp.sum(oh * tid, axis=1, keepdims=True)
    gh = gh_ref[...]
    mj = lax.broadcasted_iota(jnp.int32, (BR, 128), 1)
    rowf = (lax.broadcasted_iota(jnp.int32, (BR, 128), 0)
            + b * BR).astype(jnp.float32)
    basef = jnp.where(mj < H_NR, jnp.broadcast_to(kt, (BR, 128)), rowf)
    g = basef * float(H) + jnp.broadcast_to(gh, (BR, 128))
    g_ref[...] = g.astype(jnp.int32)


def _glist(rr, gh):
    return pl.pallas_call(
        _glist_body,
        grid=(NBR,),
        in_specs=[pl.BlockSpec((1, S), lambda b: (0, 0)),
                  pl.BlockSpec((1, 128), lambda b: (0, 0))],
        out_specs=pl.BlockSpec((BR, 128), lambda b: (b, 0)),
        out_shape=jax.ShapeDtypeStruct((S, 128), jnp.int32),
    )(rr, gh)


def _sc_gather_body(g_hbm, kv_hbm, kvout_hbm, idx_v, rows_kv, sem):
    cid = lax.axis_index("c")
    sid = lax.axis_index("s")
    wid = sid * 2 + cid
    pltpu.sync_copy(g_hbm.at[pl.ds(wid * 4, 4)], idx_v)
    for b in range(4):
        for t in range(IDX_MINOR // 16):
            v = idx_v[b, pl.ds(t * 16, 16)]
            idx_v[b, pl.ds(t * 16, 16)] = jnp.clip(v, 0, S * H - 1)
    copies = []
    for b in range(4):
        copies.append(pltpu.async_copy(
            kv_hbm.at[idx_v.at[b]],
            rows_kv.at[pl.ds(b * IDX_MINOR, IDX_MINOR)], sem))
    for cp in copies:
        cp.wait()
    pltpu.sync_copy(rows_kv, kvout_hbm.at[pl.ds(wid * ROWS_PER_TILE,
                                                ROWS_PER_TILE)])


def _sc_gather(g, kvflat):
    mesh = plsc.VectorSubcoreMesh(core_axis_name="c", subcore_axis_name="s")
    fn = functools.partial(
        pl.kernel,
        mesh=mesh,
        compiler_params=pltpu.CompilerParams(needs_layout_passes=False),
        out_type=[
            jax.ShapeDtypeStruct((N_ROWS, 2 * D), jnp.float32),
        ],
        scratch_types=[
            pltpu.VMEM((4, IDX_MINOR), jnp.int32),
            pltpu.VMEM((ROWS_PER_TILE, 2 * D), jnp.float32),
            pltpu.SemaphoreType.DMA,
        ],
    )(_sc_gather_body)
    return fn(g, kvflat)


def kernel(key_cache, value_cache, attn, comp_key_emb, comp_value_emb):
    del comp_key_emb, comp_value_emb
    a3 = attn[:, 0]
    stats, colsum = _attn_stats(a3)
    kcf = key_cache.reshape(S, H * D)
    vcf = value_cache.reshape(S, H * D)
    inv_o, gh, sn = _headscore(stats, colsum)
    scol = sn.reshape(S, 1)
    ranks_o, sumk_o, sumv_o = _rank(scol, sn, kcf, vcf)
    inv12 = inv_o[:, 0].astype(jnp.int32)
    ranks_row = ranks_o.reshape(1, S)
    g2d = _glist(ranks_row, gh)
    g_kept = g2d[:K_KEEP, :H_NR].reshape(-1)
    g_retr = g2d[:, H_NR:H].reshape(-1)
    g = jnp.concatenate([g_kept, g_retr,
                         jnp.zeros((4,), jnp.int32)]).reshape(
        N_ROWS // IDX_MINOR, IDX_MINOR)
    kvflat = jnp.concatenate([key_cache, value_cache],
                             axis=-1).reshape(S * H, 2 * D)
    (kvrows,) = _sc_gather(g, kvflat)
    krows = kvrows[:, :D]
    vrows = kvrows[:, D:]
    keys_kept = krows[:N_KEPT_ROWS].reshape(1, K_KEEP, H_NR, D)
    vals_kept = vrows[:N_KEPT_ROWS].reshape(1, K_KEEP, H_NR, D)
    retr_k = krows[N_KEPT_ROWS:N_KEPT_ROWS + N_RETR_ROWS].reshape(
        1, S, NUM_RETR, D)
    retr_v = vrows[N_KEPT_ROWS:N_KEPT_ROWS + N_RETR_ROWS].reshape(
        1, S, NUM_RETR, D)
    sumk = sumk_o.sum(axis=0).reshape(H, D)
    sumv = sumv_o.sum(axis=0).reshape(H, D)
    perm = (inv12[NUM_RETR:][:, None] == jnp.arange(H)[None, :]
            ).astype(jnp.float32)
    ck_nr = (perm @ sumk) * (1.0 / N_DROP)
    cv_nr = (perm @ sumv) * (1.0 / N_DROP)
    ck = jnp.broadcast_to(ck_nr[None, None], (1, COMP, H_NR, D))
    cv = jnp.broadcast_to(cv_nr[None, None], (1, COMP, H_NR, D))
    keys_out = jnp.concatenate([keys_kept, ck], axis=1)
    values_out = jnp.concatenate([vals_kept, cv], axis=1)
    return keys_out, values_out, retr_k, retr_v
